# baseline (speedup 1.0000x reference)
import functools
import math

import jax
import jax.numpy as jnp
from jax import lax
from jax.experimental import pallas as pl
from jax.experimental.pallas import tpu as pltpu
from jax.experimental.pallas import tpu_sc as plsc

_B = 4
_S = 4096
_D = 1024
_ROWS = _B * _S
_NW = 32
_S_PER_W = _S // _NW
_CS = 8
_N_ROUNDS = _S_PER_W // _CS
_GR = _B * _CS
_VECS_PER_ROW = _D // 16
_SCALE = math.sqrt(_D)
_NBUF = 3

_mesh = plsc.VectorSubcoreMesh(core_axis_name="c", subcore_axis_name="s")


@functools.partial(
    pl.kernel,
    out_type=jax.ShapeDtypeStruct((_ROWS, _D), jnp.float32),
    mesh=_mesh,
    scratch_types=[
        pltpu.VMEM((_B, _S_PER_W), jnp.int32),
        pltpu.VMEM((_GR, _D), jnp.float32),
        pltpu.VMEM((_GR, _D), jnp.float32),
        pltpu.VMEM((_GR, _D), jnp.float32),
        pltpu.VMEM((_CS, _D), jnp.float32),
        pltpu.VMEM((_CS, _D), jnp.float32),
        pltpu.VMEM((_CS, _D), jnp.float32),
        pltpu.SemaphoreType.DMA,
        pltpu.SemaphoreType.DMA,
        pltpu.SemaphoreType.DMA,
        pltpu.SemaphoreType.DMA,
        pltpu.SemaphoreType.DMA,
        pltpu.SemaphoreType.DMA,
        pltpu.SemaphoreType.DMA,
        pltpu.SemaphoreType.DMA,
        pltpu.SemaphoreType.DMA,
    ],
)
def _embed(tokens_hbm, table_hbm, pos_hbm, out_hbm, idx_all,
           stg0, stg1, stg2, pos0, pos1, pos2,
           gsem0, gsem1, gsem2, psem0, psem1, psem2,
           osem0, osem1, osem2):
    wid = lax.axis_index("s") * 2 + lax.axis_index("c")
    s0 = wid * _S_PER_W

    stg = (stg0, stg1, stg2)
    pos = (pos0, pos1, pos2)
    gsem = (gsem0, gsem1, gsem2)
    psem = (psem0, psem1, psem2)
    osem = (osem0, osem1, osem2)

    for b in range(_B):
        pltpu.sync_copy(tokens_hbm.at[pl.ds(b * _S + s0, _S_PER_W)],
                        idx_all.at[b])

    def issue(r, j):
        for b in range(_B):
            idx_sl = idx_all.at[b, pl.ds(r * _CS, _CS)]
            pltpu.async_copy(table_hbm.at[idx_sl],
                             stg[j].at[pl.ds(b * _CS, _CS)], gsem[j])
        pltpu.async_copy(pos_hbm.at[pl.ds(s0 + r * _CS, _CS)], pos[j], psem[j])

    def do_round(r, j, tail):
        pltpu.make_async_copy(table_hbm.at[idx_all.at[0, pl.ds(0, _GR)]],
                              stg[j], gsem[j]).wait()
        pltpu.make_async_copy(pos_hbm.at[pl.ds(0, _CS)], pos[j],
                              psem[j]).wait()

        def vec_body(v, _):
            col = v * 16
            for i in range(_CS):
                ps = pos[j][i, pl.ds(col, 16)] * _SCALE
                for b in range(_B):
                    row = b * _CS + i
                    stg[j][row, pl.ds(col, 16)] = (
                        stg[j][row, pl.ds(col, 16)] * _SCALE + ps
                    )
            return 0

        lax.fori_loop(0, _VECS_PER_ROW, vec_body, 0)

        if not tail:
            jn = (j + 2) % _NBUF

            @pl.when(jnp.logical_and(r >= 1, r + 2 < _N_ROUNDS))
            def _():
                pltpu.make_async_copy(
                    stg[jn], out_hbm.at[pl.ds(0, _GR)], osem[jn]).wait()

            @pl.when(r + 2 < _N_ROUNDS)
            def _():
                issue(r + 2, jn)

        for b in range(_B):
            pltpu.async_copy(
                stg[j].at[pl.ds(b * _CS, _CS)],
                out_hbm.at[pl.ds(b * _S + s0 + r * _CS, _CS)],
                osem[j])

    issue(0, 0)
    issue(1, 1)

    def block_body(g, _):
        r0 = g * _NBUF
        for k in range(_NBUF):
            do_round(r0 + k, k, tail=False)
        return 0

    lax.fori_loop(0, (_N_ROUNDS - 1) // _NBUF, block_body, 0)
    do_round(_N_ROUNDS - 1, (_N_ROUNDS - 1) % _NBUF, tail=True)

    for r in range(_N_ROUNDS - _NBUF, _N_ROUNDS):
        j = r % _NBUF
        pltpu.make_async_copy(stg[j], out_hbm.at[pl.ds(0, _GR)],
                              osem[j]).wait()


def kernel(tokens, token_table, pos_table):
    batch, seq = tokens.shape
    flat = tokens.reshape(-1).astype(jnp.int32)
    out = _embed(flat, token_table, pos_table)
    return out.reshape(batch, seq, _D)


# speedup vs baseline: 1.0102x; 1.0102x over previous
import functools
import math

import jax
import jax.numpy as jnp
from jax import lax
from jax.experimental import pallas as pl
from jax.experimental.pallas import tpu as pltpu
from jax.experimental.pallas import tpu_sc as plsc

_B = 4
_S = 4096
_D = 1024
_ROWS = _B * _S
_NW = 32
_S_PER_W = _S // _NW
_CS = 8
_N_ROUNDS = _S_PER_W // _CS
_GR = _B * _CS
_VECS_PER_ROW = _D // 16
_SCALE = math.sqrt(_D)
_NBUF = 3

_mesh = plsc.VectorSubcoreMesh(core_axis_name="c", subcore_axis_name="s")


@functools.partial(
    pl.kernel,
    out_type=jax.ShapeDtypeStruct((_ROWS, _D), jnp.float32),
    mesh=_mesh,
    scratch_types=[
        pltpu.VMEM((_N_ROUNDS * _GR,), jnp.int32),
        pltpu.VMEM((_GR, _D), jnp.float32),
        pltpu.VMEM((_GR, _D), jnp.float32),
        pltpu.VMEM((_GR, _D), jnp.float32),
        pltpu.VMEM((_CS, _D), jnp.float32),
        pltpu.VMEM((_CS, _D), jnp.float32),
        pltpu.VMEM((_CS, _D), jnp.float32),
        pltpu.SemaphoreType.DMA,
        pltpu.SemaphoreType.DMA,
        pltpu.SemaphoreType.DMA,
        pltpu.SemaphoreType.DMA,
        pltpu.SemaphoreType.DMA,
        pltpu.SemaphoreType.DMA,
        pltpu.SemaphoreType.DMA,
        pltpu.SemaphoreType.DMA,
        pltpu.SemaphoreType.DMA,
    ],
)
def _embed(perm_hbm, table_hbm, pos_hbm, out_hbm, idx_all,
           stg0, stg1, stg2, pos0, pos1, pos2,
           gsem0, gsem1, gsem2, psem0, psem1, psem2,
           osem0, osem1, osem2):
    wid = lax.axis_index("s") * 2 + lax.axis_index("c")
    s0 = wid * _S_PER_W

    stg = (stg0, stg1, stg2)
    pos = (pos0, pos1, pos2)
    gsem = (gsem0, gsem1, gsem2)
    psem = (psem0, psem1, psem2)
    osem = (osem0, osem1, osem2)

    pltpu.sync_copy(perm_hbm.at[wid], idx_all)

    def issue(r, j):
        idx_sl = idx_all.at[pl.ds(r * _GR, _GR)]
        pltpu.async_copy(table_hbm.at[idx_sl], stg[j], gsem[j])
        pltpu.async_copy(pos_hbm.at[pl.ds(s0 + r * _CS, _CS)], pos[j], psem[j])

    def do_round(r, j, tail):
        idx_sl = idx_all.at[pl.ds(r * _GR, _GR)]
        pltpu.make_async_copy(table_hbm.at[idx_sl], stg[j], gsem[j]).wait()
        pltpu.make_async_copy(pos_hbm.at[pl.ds(0, _CS)], pos[j],
                              psem[j]).wait()

        def vec_body(v, _):
            col = v * 16
            for i in range(_CS):
                ps = pos[j][i, pl.ds(col, 16)] * _SCALE
                for b in range(_B):
                    row = b * _CS + i
                    stg[j][row, pl.ds(col, 16)] = (
                        stg[j][row, pl.ds(col, 16)] * _SCALE + ps
                    )
            return 0

        lax.fori_loop(0, _VECS_PER_ROW // 2, vec_body, 0)

        if not tail:
            jn = (j + 2) % _NBUF

            @pl.when(jnp.logical_and(r >= 1, r + 2 < _N_ROUNDS))
            def _():
                pltpu.make_async_copy(
                    stg[jn], out_hbm.at[pl.ds(0, _GR)], osem[jn]).wait()

            @pl.when(r + 2 < _N_ROUNDS)
            def _():
                issue(r + 2, jn)

        lax.fori_loop(_VECS_PER_ROW // 2, _VECS_PER_ROW, vec_body, 0)

        for b in range(_B):
            pltpu.async_copy(
                stg[j].at[pl.ds(b * _CS, _CS)],
                out_hbm.at[pl.ds(b * _S + s0 + r * _CS, _CS)],
                osem[j])

    issue(0, 0)
    issue(1, 1)

    def block_body(g, _):
        r0 = g * _NBUF
        for k in range(_NBUF):
            do_round(r0 + k, k, tail=False)
        return 0

    lax.fori_loop(0, (_N_ROUNDS - 1) // _NBUF, block_body, 0)
    do_round(_N_ROUNDS - 1, (_N_ROUNDS - 1) % _NBUF, tail=True)

    for r in range(_N_ROUNDS - _NBUF, _N_ROUNDS):
        j = r % _NBUF
        pltpu.make_async_copy(stg[j], out_hbm.at[pl.ds(0, _GR)],
                              osem[j]).wait()


def kernel(tokens, token_table, pos_table):
    batch, seq = tokens.shape
    perm = (
        tokens.astype(jnp.int32)
        .reshape(batch, _NW, _N_ROUNDS, _CS)
        .transpose(1, 2, 0, 3)
        .reshape(_NW, _N_ROUNDS * _GR)
    )
    out = _embed(perm, token_table, pos_table)
    return out.reshape(batch, seq, _D)


# speedup vs baseline: 1.0113x; 1.0011x over previous
import functools
import math

import jax
import jax.numpy as jnp
from jax import lax
from jax.experimental import pallas as pl
from jax.experimental.pallas import tpu as pltpu
from jax.experimental.pallas import tpu_sc as plsc

_B = 4
_S = 4096
_D = 1024
_ROWS = _B * _S
_NW = 32
_S_PER_W = _S // _NW
_CS = 8
_N_ROUNDS = _S_PER_W // _CS
_GR = _B * _CS
_VECS_PER_ROW = _D // 16
_SCALE = math.sqrt(_D)
_NBUF = 3

_mesh = plsc.VectorSubcoreMesh(core_axis_name="c", subcore_axis_name="s")


@functools.partial(
    pl.kernel,
    out_type=jax.ShapeDtypeStruct((_ROWS, _D), jnp.float32),
    mesh=_mesh,
    scratch_types=[
        pltpu.VMEM((_N_ROUNDS * _GR,), jnp.int32),
        pltpu.VMEM((_GR, _D), jnp.float32),
        pltpu.VMEM((_GR, _D), jnp.float32),
        pltpu.VMEM((_GR, _D), jnp.float32),
        pltpu.VMEM((_CS, _D), jnp.float32),
        pltpu.VMEM((_CS, _D), jnp.float32),
        pltpu.VMEM((_CS, _D), jnp.float32),
        pltpu.SemaphoreType.DMA,
        pltpu.SemaphoreType.DMA,
        pltpu.SemaphoreType.DMA,
        pltpu.SemaphoreType.DMA,
        pltpu.SemaphoreType.DMA,
        pltpu.SemaphoreType.DMA,
        pltpu.SemaphoreType.DMA,
        pltpu.SemaphoreType.DMA,
        pltpu.SemaphoreType.DMA,
    ],
)
def _embed(perm_hbm, table_hbm, pos_hbm, out_hbm, idx_all,
           stg0, stg1, stg2, pos0, pos1, pos2,
           gsem0, gsem1, gsem2, psem0, psem1, psem2,
           osem0, osem1, osem2):
    wid = lax.axis_index("s") * 2 + lax.axis_index("c")
    s0 = wid * _S_PER_W

    stg = (stg0, stg1, stg2)
    pos = (pos0, pos1, pos2)
    gsem = (gsem0, gsem1, gsem2)
    psem = (psem0, psem1, psem2)
    osem = (osem0, osem1, osem2)

    pltpu.sync_copy(perm_hbm.at[wid], idx_all)

    def issue(r, j):
        idx_sl = idx_all.at[pl.ds(r * _GR, _GR)]
        pltpu.async_copy(table_hbm.at[idx_sl], stg[j], gsem[j])
        pltpu.async_copy(pos_hbm.at[pl.ds(s0 + r * _CS, _CS)], pos[j], psem[j])

    def do_round(r, j, tail):
        idx_sl = idx_all.at[pl.ds(r * _GR, _GR)]
        pltpu.make_async_copy(table_hbm.at[idx_sl], stg[j], gsem[j]).wait()
        pltpu.make_async_copy(pos_hbm.at[pl.ds(0, _CS)], pos[j],
                              psem[j]).wait()

        def vec_body(v, _):
            col = v * 16
            for i in range(_CS):
                ps = pos[j][i, pl.ds(col, 16)] * _SCALE
                for b in range(_B):
                    row = b * _CS + i
                    stg[j][row, pl.ds(col, 16)] = (
                        stg[j][row, pl.ds(col, 16)] * _SCALE + ps
                    )
            return 0

        lax.fori_loop(0, _VECS_PER_ROW, vec_body, 0)

        if not tail:
            jn = (j + 2) % _NBUF

            @pl.when(jnp.logical_and(r >= 1, r + 2 < _N_ROUNDS))
            def _():
                pltpu.make_async_copy(
                    stg[jn], out_hbm.at[pl.ds(0, _GR)], osem[jn]).wait()
                issue(r + 2, jn)

        for b in range(_B):
            pltpu.async_copy(
                stg[j].at[pl.ds(b * _CS, _CS)],
                out_hbm.at[pl.ds(b * _S + s0 + r * _CS, _CS)],
                osem[j])

    issue(0, 0)
    issue(1, 1)
    issue(2, 2)

    def block_body(g, _):
        r0 = g * _NBUF
        for k in range(_NBUF):
            do_round(r0 + k, k, tail=False)
        return 0

    lax.fori_loop(0, (_N_ROUNDS - 1) // _NBUF, block_body, 0)
    do_round(_N_ROUNDS - 1, (_N_ROUNDS - 1) % _NBUF, tail=True)

    for r in range(_N_ROUNDS - _NBUF, _N_ROUNDS):
        j = r % _NBUF
        pltpu.make_async_copy(stg[j], out_hbm.at[pl.ds(0, _GR)],
                              osem[j]).wait()


def kernel(tokens, token_table, pos_table):
    batch, seq = tokens.shape
    perm = (
        tokens.astype(jnp.int32)
        .reshape(batch, _NW, _N_ROUNDS, _CS)
        .transpose(1, 2, 0, 3)
        .reshape(_NW, _N_ROUNDS * _GR)
    )
    out = _embed(perm, token_table, pos_table)
    return out.reshape(batch, seq, _D)


# speedup vs baseline: 1.0216x; 1.0101x over previous
import functools
import math

import jax
import jax.numpy as jnp
from jax import lax
from jax.experimental import pallas as pl
from jax.experimental.pallas import tpu as pltpu
from jax.experimental.pallas import tpu_sc as plsc

_B = 4
_S = 4096
_D = 1024
_ROWS = _B * _S
_NW = 32
_S_PER_W = _S // _NW
_CS = 8
_N_ROUNDS = _S_PER_W // _CS
_GR = _B * _CS
_VECS_PER_ROW = _D // 16
_SCALE = math.sqrt(_D)
_NBUF = 3

_mesh = plsc.VectorSubcoreMesh(core_axis_name="c", subcore_axis_name="s")


@functools.partial(
    pl.kernel,
    out_type=jax.ShapeDtypeStruct((_ROWS, _D), jnp.float32),
    mesh=_mesh,
    scratch_types=[
        pltpu.VMEM((_N_ROUNDS * _GR,), jnp.int32),
        pltpu.VMEM((_GR, _D), jnp.float32),
        pltpu.VMEM((_GR, _D), jnp.float32),
        pltpu.VMEM((_GR, _D), jnp.float32),
        pltpu.VMEM((_CS, _D), jnp.float32),
        pltpu.VMEM((_CS, _D), jnp.float32),
        pltpu.VMEM((_CS, _D), jnp.float32),
        pltpu.SemaphoreType.DMA,
        pltpu.SemaphoreType.DMA,
        pltpu.SemaphoreType.DMA,
        pltpu.SemaphoreType.DMA,
        pltpu.SemaphoreType.DMA,
        pltpu.SemaphoreType.DMA,
        pltpu.SemaphoreType.DMA,
        pltpu.SemaphoreType.DMA,
        pltpu.SemaphoreType.DMA,
    ],
)
def _embed(perm_hbm, table_hbm, pos_hbm, out_hbm, idx_all,
           stg0, stg1, stg2, pos0, pos1, pos2,
           gsem0, gsem1, gsem2, psem0, psem1, psem2,
           osem0, osem1, osem2):
    wid = lax.axis_index("s") * 2 + lax.axis_index("c")
    s0 = wid * _S_PER_W

    stg = (stg0, stg1, stg2)
    pos = (pos0, pos1, pos2)
    gsem = (gsem0, gsem1, gsem2)
    psem = (psem0, psem1, psem2)
    osem = (osem0, osem1, osem2)

    pltpu.sync_copy(perm_hbm.at[wid], idx_all)

    def issue(r, j):
        idx_sl = idx_all.at[pl.ds(r * _GR, _GR)]
        pltpu.async_copy(table_hbm.at[idx_sl], stg[j], gsem[j])
        pltpu.async_copy(pos_hbm.at[pl.ds(s0 + r * _CS, _CS)], pos[j], psem[j])

    def do_round(r, j, tail):
        idx_sl = idx_all.at[pl.ds(r * _GR, _GR)]
        pltpu.make_async_copy(table_hbm.at[idx_sl], stg[j], gsem[j]).wait()
        pltpu.make_async_copy(pos_hbm.at[pl.ds(0, _CS)], pos[j],
                              psem[j]).wait()

        def vec_body(v, _):
            col = v * 16
            for i in range(_CS):
                ps = pos[j][i, pl.ds(col, 16)] * _SCALE
                for b in range(_B):
                    row = b * _CS + i
                    stg[j][row, pl.ds(col, 16)] = (
                        stg[j][row, pl.ds(col, 16)] * _SCALE + ps
                    )
            return 0

        lax.fori_loop(0, _VECS_PER_ROW, vec_body, 0)

        if not tail:
            jn = (j + 2) % _NBUF

            @pl.when(jnp.logical_and(r >= 1, r + 2 < _N_ROUNDS))
            def _():
                pltpu.make_async_copy(
                    stg[jn], out_hbm.at[pl.ds(0, _GR)], osem[jn]).wait()

            @pl.when(r + 2 < _N_ROUNDS)
            def _():
                issue(r + 2, jn)

        for b in range(_B):
            pltpu.async_copy(
                stg[j].at[pl.ds(b * _CS, _CS)],
                out_hbm.at[pl.ds(b * _S + s0 + r * _CS, _CS)],
                osem[j])

    issue(0, 0)
    issue(1, 1)

    def block_body(g, _):
        r0 = g * _NBUF
        for k in range(_NBUF):
            do_round(r0 + k, k, tail=False)
        return 0

    lax.fori_loop(0, (_N_ROUNDS - 1) // _NBUF, block_body, 0)
    do_round(_N_ROUNDS - 1, (_N_ROUNDS - 1) % _NBUF, tail=True)

    for r in range(_N_ROUNDS - _NBUF, _N_ROUNDS):
        j = r % _NBUF
        pltpu.make_async_copy(stg[j], out_hbm.at[pl.ds(0, _GR)],
                              osem[j]).wait()


def kernel(tokens, token_table, pos_table):
    batch, seq = tokens.shape
    perm = (
        tokens.astype(jnp.int32)
        .reshape(batch, _NW, _N_ROUNDS, _CS)
        .transpose(1, 2, 0, 3)
        .reshape(_NW, _N_ROUNDS * _GR)
    )
    out = _embed(perm, token_table, pos_table)
    return out.reshape(batch, seq, _D)
